# baseline (speedup 1.0000x reference)
import functools

import jax
import jax.numpy as jnp
from jax.experimental import pallas as pl
from jax.experimental.pallas import tpu as pltpu

_TMPL_W = 31360


def _mask_body(nsel, ncols, allowed_ref, scores_hbm, out_hbm, tmpl_ref, gath_ref, fix_ref, gsem, fsem, xsem):
    nrows = out_hbm.shape[0]
    nfull = ncols // _TMPL_W
    rem = ncols - nfull * _TMPL_W
    rem128 = (rem // 128) * 128
    tail = rem - rem128

    def gather_copy(k):
        src = pl.multiple_of((allowed_ref[k] // 128) * 128, 128)
        return pltpu.make_async_copy(
            scores_hbm.at[:, pl.ds(src, 128)],
            gath_ref.at[:, pl.ds(k * 128, 128)],
            gsem,
        )

    def fill_copies():
        copies = [
            pltpu.make_async_copy(
                tmpl_ref, out_hbm.at[:, pl.ds(b * _TMPL_W, _TMPL_W)], fsem
            )
            for b in range(nfull)
        ]
        if rem128:
            copies.append(
                pltpu.make_async_copy(
                    tmpl_ref.at[:, pl.ds(0, rem128)],
                    out_hbm.at[:, pl.ds(nfull * _TMPL_W, rem128)],
                    fsem,
                )
            )
        if tail:
            tb = pl.multiple_of(allowed_ref[nsel], 128)
            copies.append(
                pltpu.make_async_copy(
                    tmpl_ref.at[:, pl.ds(0, 128)],
                    out_hbm.at[:, pl.ds(tb, 128)],
                    fsem,
                )
            )
        return copies

    def fix_copy(k):
        dst = pl.multiple_of((allowed_ref[k] // 128) * 128, 128)
        return pltpu.make_async_copy(
            fix_ref.at[:, pl.ds(k * 128, 128)],
            out_hbm.at[:, pl.ds(dst, 128)],
            xsem,
        )

    for k in range(nsel):
        gather_copy(k).start()

    tmpl_ref[...] = jnp.full(tmpl_ref.shape, -jnp.inf, tmpl_ref.dtype)
    for c in fill_copies():
        c.start()

    for k in range(nsel):
        gather_copy(k).wait()

    lanes = jax.lax.broadcasted_iota(jnp.int32, (nrows, 128), 1)
    for k in range(nsel):
        gath = gath_ref[:, k * 128 : (k + 1) * 128]
        fix_ref[:, k * 128 : (k + 1) * 128] = jnp.where(
            lanes == allowed_ref[k] % 128, gath, -jnp.inf
        )

    for c in fill_copies():
        c.wait()
    for k in range(nsel):
        fix_copy(k).start()
    for k in range(nsel):
        fix_copy(k).wait()


def kernel(input_ids, scores, allowed):
    nrows, ncols = scores.shape
    nsel = allowed.shape[0]
    tail_base = jnp.array([(ncols // 128) * 128], dtype=allowed.dtype)
    scal = jnp.concatenate([allowed, tail_base])
    grid_spec = pltpu.PrefetchScalarGridSpec(
        num_scalar_prefetch=1,
        grid=(1,),
        in_specs=[pl.BlockSpec(memory_space=pl.ANY)],
        out_specs=pl.BlockSpec(memory_space=pl.ANY),
        scratch_shapes=[
            pltpu.VMEM((nrows, _TMPL_W), jnp.float32),
            pltpu.VMEM((nrows, nsel * 128), jnp.float32),
            pltpu.VMEM((nrows, nsel * 128), jnp.float32),
            pltpu.SemaphoreType.DMA,
            pltpu.SemaphoreType.DMA,
            pltpu.SemaphoreType.DMA,
        ],
    )
    return pl.pallas_call(
        functools.partial(_mask_body, nsel, ncols),
        grid_spec=grid_spec,
        out_shape=jax.ShapeDtypeStruct(scores.shape, scores.dtype),
    )(scal, scores)


# speedup vs baseline: 1.0405x; 1.0405x over previous
import functools

import jax
import jax.numpy as jnp
from jax.experimental import pallas as pl
from jax.experimental.pallas import tpu as pltpu

_LANE_BLOCK = 31360


def _mask_body(nsel, allowed_ref, scores_hbm, out_ref, gath_ref, sem):
    i = pl.program_id(0)
    base = i * _LANE_BLOCK
    nrows = out_ref.shape[0]

    def col_copy(k):
        src = pl.multiple_of((allowed_ref[k] // 128) * 128, 128)
        return pltpu.make_async_copy(
            scores_hbm.at[:, pl.ds(src, 128)],
            gath_ref.at[:, pl.ds(k * 128, 128)],
            sem,
        )

    @pl.when(i == 0)
    def _():
        for k in range(nsel):
            col_copy(k).start()

    out_ref[...] = jnp.full(out_ref.shape, -jnp.inf, out_ref.dtype)

    @pl.when(i == 0)
    def _():
        for k in range(nsel):
            col_copy(k).wait()

    lanes = jax.lax.broadcasted_iota(jnp.int32, (nrows, 128), 1)
    for k in range(nsel):
        off = allowed_ref[k] - base

        @pl.when((off >= 0) & (off < _LANE_BLOCK))
        def _():
            chunk_base = pl.multiple_of((off // 128) * 128, 128)
            chunk = out_ref[:, pl.ds(chunk_base, 128)]
            gath = gath_ref[:, k * 128 : (k + 1) * 128]
            chunk = jnp.where(lanes == off % 128, gath, chunk)
            out_ref[:, pl.ds(chunk_base, 128)] = chunk


def kernel(input_ids, scores, allowed):
    nrows, ncols = scores.shape
    nsel = allowed.shape[0]
    grid = pl.cdiv(ncols, _LANE_BLOCK)
    grid_spec = pltpu.PrefetchScalarGridSpec(
        num_scalar_prefetch=1,
        grid=(grid,),
        in_specs=[pl.BlockSpec(memory_space=pl.ANY)],
        out_specs=pl.BlockSpec((nrows, _LANE_BLOCK), lambda i, a: (0, i)),
        scratch_shapes=[
            pltpu.VMEM((nrows, nsel * 128), jnp.float32),
            pltpu.SemaphoreType.DMA,
        ],
    )
    return pl.pallas_call(
        functools.partial(_mask_body, nsel),
        grid_spec=grid_spec,
        out_shape=jax.ShapeDtypeStruct(scores.shape, scores.dtype),
    )(allowed, scores)
